# Initial kernel scaffold; baseline (speedup 1.0000x reference)
import jax
import jax.numpy as jnp
from jax.experimental import pallas as pl


def kernel(obs, edge_index, gcn_W1, gcn_b1, gcn_W2, gcn_b2, lin_W, lin_b, a1_W, a1_b, a2_W, a2_b, c1_W, c1_b, c2_W, c2_b):
    raise NotImplementedError("write your pallas kernel here")


# speedup vs baseline: 24.8560x; 24.8560x over previous
import functools

import jax
import jax.numpy as jnp
from jax import lax
from jax.experimental import pallas as pl
from jax.experimental.pallas import tpu as pltpu
from jax.experimental.pallas import tpu_sc as plsc

N = 10000
E = 320000
C = 128
NC = 2
NS = 16
NW = NC * NS
CHUNK = 128
NCHUNK = 80
SUP = 10
NSUP = NCHUNK // SUP
EPW = NCHUNK * CHUNK
EP = NW * EPW
NPAD = 10240
RPS = NPAD // NS
TRASH = NPAD - N

_mesh = plsc.VectorSubcoreMesh(
    core_axis_name="c", subcore_axis_name="s", num_cores=NC, num_subcores=NS)


def _worker(c, s):
    return c * NS + s


@functools.partial(
    pl.kernel,
    out_type=jax.ShapeDtypeStruct((NC * NPAD,), jnp.float32),
    mesh=_mesh,
    scratch_types=[
        pltpu.VMEM((NCHUNK, CHUNK), jnp.int32),
        pltpu.VMEM((CHUNK,), jnp.float32),
        pltpu.VMEM_SHARED((NPAD,), jnp.float32),
    ],
)
def _deg_kernel(dst_hbm, ones_hbm, zeros_hbm, out_hbm, idx_v, ones_v,
                deg_shared):
    c = lax.axis_index("c")
    s = lax.axis_index("s")
    w = _worker(c, s)
    pltpu.sync_copy(zeros_hbm, deg_shared.at[pl.ds(s * RPS, RPS)])
    pltpu.sync_copy(dst_hbm.at[w], idx_v)
    pltpu.sync_copy(ones_hbm, ones_v)
    plsc.subcore_barrier()

    def body(j, carry):
        pltpu.sync_copy(ones_v, deg_shared.at[idx_v.at[j]], add=True)
        return carry
    lax.fori_loop(0, NCHUNK, body, 0)
    plsc.subcore_barrier()
    pltpu.sync_copy(deg_shared.at[pl.ds(s * RPS, RPS)],
                    out_hbm.at[pl.ds(c * NPAD + s * RPS, RPS)])


def _make_agg_kernel(width):
    @functools.partial(
        pl.kernel,
        out_type=jax.ShapeDtypeStruct((NC * NPAD, width), jnp.float32),
        mesh=_mesh,
        scratch_types=[
            pltpu.VMEM((SUP, CHUNK), jnp.int32),
            pltpu.VMEM((SUP, CHUNK), jnp.int32),
            pltpu.VMEM((2, CHUNK, width), jnp.float32),
            pltpu.VMEM_SHARED((NPAD, width), jnp.float32),
            pltpu.SemaphoreType.DMA,
        ],
    )
    def agg(y_hbm, src_hbm, dst_hbm, out_hbm, src_v, dst_v, rows_v,
            acc_shared, gsem):
        c = lax.axis_index("c")
        s = lax.axis_index("s")
        w = _worker(c, s)
        pltpu.sync_copy(y_hbm.at[pl.ds(s * RPS, RPS)],
                        acc_shared.at[pl.ds(s * RPS, RPS)])
        plsc.subcore_barrier()

        def outer(g, carry):
            pltpu.sync_copy(src_hbm.at[w * NSUP + g], src_v)
            pltpu.sync_copy(dst_hbm.at[w * NSUP + g], dst_v)
            pltpu.async_copy(y_hbm.at[src_v.at[0]], rows_v.at[0], gsem)

            def body(j, carry2):
                nxt = j + 1
                @pl.when(nxt < SUP)
                def _start():
                    pltpu.async_copy(y_hbm.at[src_v.at[nxt]],
                                     rows_v.at[lax.rem(nxt, 2)], gsem)
                pltpu.make_async_copy(y_hbm.at[src_v.at[j]],
                                      rows_v.at[lax.rem(j, 2)], gsem).wait()
                pltpu.sync_copy(rows_v.at[lax.rem(j, 2)],
                                acc_shared.at[dst_v.at[j]], add=True)
                return carry2
            lax.fori_loop(0, SUP, body, 0)
            return carry
        lax.fori_loop(0, NSUP, outer, 0)
        plsc.subcore_barrier()
        pltpu.sync_copy(acc_shared.at[pl.ds(s * RPS, RPS)],
                        out_hbm.at[pl.ds(c * NPAD + s * RPS, RPS)])
    return agg


_agg128 = _make_agg_kernel(C)


@functools.partial(
    pl.kernel,
    out_type=jax.ShapeDtypeStruct((NC * 2 * NPAD,), jnp.float32),
    mesh=_mesh,
    scratch_types=[
        pltpu.VMEM((SUP, CHUNK), jnp.int32),
        pltpu.VMEM((SUP, CHUNK), jnp.int32),
        pltpu.VMEM((CHUNK,), jnp.float32),
        pltpu.VMEM((CHUNK,), jnp.float32),
        pltpu.VMEM_SHARED((NPAD,), jnp.float32),
        pltpu.VMEM_SHARED((NPAD,), jnp.float32),
    ],
)
def _agg2(y0_hbm, y1_hbm, src_hbm, dst_hbm, out_hbm, src_v, dst_v,
          v0, v1, acc0_sh, acc1_sh):
    c = lax.axis_index("c")
    s = lax.axis_index("s")
    w = _worker(c, s)
    pltpu.sync_copy(y0_hbm.at[pl.ds(s * RPS, RPS)],
                    acc0_sh.at[pl.ds(s * RPS, RPS)])
    pltpu.sync_copy(y1_hbm.at[pl.ds(s * RPS, RPS)],
                    acc1_sh.at[pl.ds(s * RPS, RPS)])
    plsc.subcore_barrier()

    def outer(g, carry):
        pltpu.sync_copy(src_hbm.at[w * NSUP + g], src_v)
        pltpu.sync_copy(dst_hbm.at[w * NSUP + g], dst_v)

        def body(j, carry2):
            pltpu.sync_copy(y0_hbm.at[src_v.at[j]], v0)
            pltpu.sync_copy(y1_hbm.at[src_v.at[j]], v1)
            pltpu.sync_copy(v0, acc0_sh.at[dst_v.at[j]], add=True)
            pltpu.sync_copy(v1, acc1_sh.at[dst_v.at[j]], add=True)
            return carry2
        lax.fori_loop(0, SUP, body, 0)
        return carry
    lax.fori_loop(0, NSUP, outer, 0)
    plsc.subcore_barrier()
    pltpu.sync_copy(acc0_sh.at[pl.ds(s * RPS, RPS)],
                    out_hbm.at[pl.ds(c * 2 * NPAD + s * RPS, RPS)])
    pltpu.sync_copy(acc1_sh.at[pl.ds(s * RPS, RPS)],
                    out_hbm.at[pl.ds(c * 2 * NPAD + NPAD + s * RPS, RPS)])


_BLK = 512
_GRID = NPAD // _BLK


def _tc1_body(x_ref, w1_ref, degp_ref, y1_ref, dinv_ref):
    deg = degp_ref[0, :] + degp_ref[1, :] + 1.0
    dinv = 1.0 / jnp.sqrt(deg)
    h = jnp.dot(x_ref[...], w1_ref[...], preferred_element_type=jnp.float32)
    y1_ref[...] = h * dinv[:, None]
    dinv_ref[...] = dinv


def _tc1(x, w1, degp):
    return pl.pallas_call(
        _tc1_body,
        grid=(_GRID,),
        in_specs=[
            pl.BlockSpec((_BLK, C), lambda i: (i, 0)),
            pl.BlockSpec((C, C), lambda i: (0, 0)),
            pl.BlockSpec((NC, _BLK), lambda i: (0, i)),
        ],
        out_specs=[
            pl.BlockSpec((_BLK, C), lambda i: (i, 0)),
            pl.BlockSpec((_BLK,), lambda i: (i,)),
        ],
        out_shape=[
            jax.ShapeDtypeStruct((NPAD, C), jnp.float32),
            jax.ShapeDtypeStruct((NPAD,), jnp.float32),
        ],
    )(x, w1, degp)


def _tc2_body(accp_ref, y1_ref, dinv_ref, b1_ref, w2_ref, y2t_ref):
    agg = accp_ref[0] + accp_ref[1] - y1_ref[...]
    dinv = dinv_ref[...]
    out1 = jax.nn.relu(agg * dinv[:, None] + b1_ref[...][None, :])
    h2t = lax.dot_general(w2_ref[...], out1, (((0,), (1,)), ((), ())),
                          preferred_element_type=jnp.float32)
    y2t_ref[...] = h2t * dinv[None, :]


def _tc2(accp, y1, dinv, b1, w2):
    return pl.pallas_call(
        _tc2_body,
        grid=(_GRID,),
        in_specs=[
            pl.BlockSpec((NC, _BLK, C), lambda i: (0, i, 0)),
            pl.BlockSpec((_BLK, C), lambda i: (i, 0)),
            pl.BlockSpec((_BLK,), lambda i: (i,)),
            pl.BlockSpec((C,), lambda i: (0,)),
            pl.BlockSpec((C, 2), lambda i: (0, 0)),
        ],
        out_specs=pl.BlockSpec((2, _BLK), lambda i: (0, i)),
        out_shape=jax.ShapeDtypeStruct((2, NPAD), jnp.float32),
    )(accp, y1, dinv, b1, w2)


def _tc3_body(acc2p_ref, y2_ref, dinv_ref, b2_ref, linw_ref, linb_ref,
              a1w_ref, a1b_ref, a2w_ref, a2b_ref, c1w_ref, c1b_ref,
              c2w_ref, c2b_ref, am_ref, sv_ref):
    agg = (acc2p_ref[0] + acc2p_ref[1] - y2_ref[...]).T
    dinv = dinv_ref[...]
    out2 = jax.nn.relu(agg * dinv[:, None] + b2_ref[...][None, :])
    feats = jnp.dot(out2, linw_ref[...],
                    preferred_element_type=jnp.float32) + linb_ref[...][None, :]
    a = jax.nn.relu(jnp.dot(feats, a1w_ref[...],
                            preferred_element_type=jnp.float32)
                    + a1b_ref[...][None, :])
    logits = jnp.dot(a, a2w_ref[...],
                     preferred_element_type=jnp.float32) + a2b_ref[...][None, :]
    m = jnp.max(logits, axis=-1, keepdims=True)
    ex = jnp.exp(logits - m)
    am_ref[...] = ex / jnp.sum(ex, axis=-1, keepdims=True)
    cv = jax.nn.relu(jnp.dot(feats, c1w_ref[...],
                             preferred_element_type=jnp.float32)
                     + c1b_ref[...][None, :])
    sv_ref[...] = jnp.dot(cv, c2w_ref[...],
                          preferred_element_type=jnp.float32) + c2b_ref[...][None, :]


def _tc3(acc2p, y2, dinv, b2, lin_w, lin_b, a1w, a1b, a2w, a2b, c1w, c1b,
         c2w, c2b):
    return pl.pallas_call(
        _tc3_body,
        grid=(_GRID,),
        in_specs=[
            pl.BlockSpec((NC, 2, _BLK), lambda i: (0, 0, i)),
            pl.BlockSpec((2, _BLK), lambda i: (0, i)),
            pl.BlockSpec((_BLK,), lambda i: (i,)),
            pl.BlockSpec((2,), lambda i: (0,)),
            pl.BlockSpec((2, 2), lambda i: (0, 0)),
            pl.BlockSpec((2,), lambda i: (0,)),
            pl.BlockSpec((2, C), lambda i: (0, 0)),
            pl.BlockSpec((C,), lambda i: (0,)),
            pl.BlockSpec((C, 10), lambda i: (0, 0)),
            pl.BlockSpec((10,), lambda i: (0,)),
            pl.BlockSpec((2, C), lambda i: (0, 0)),
            pl.BlockSpec((C,), lambda i: (0,)),
            pl.BlockSpec((C, 1), lambda i: (0, 0)),
            pl.BlockSpec((1,), lambda i: (0,)),
        ],
        out_specs=[
            pl.BlockSpec((_BLK, 10), lambda i: (i, 0)),
            pl.BlockSpec((_BLK, 1), lambda i: (i, 0)),
        ],
        out_shape=[
            jax.ShapeDtypeStruct((NPAD, 10), jnp.float32),
            jax.ShapeDtypeStruct((NPAD, 1), jnp.float32),
        ],
    )(acc2p, y2, dinv, b2, lin_w, lin_b, a1w, a1b, a2w, a2b, c1w, c1b,
      c2w, c2b)


def kernel(obs, edge_index, gcn_W1, gcn_b1, gcn_W2, gcn_b2, lin_W, lin_b,
           a1_W, a1_b, a2_W, a2_b, c1_W, c1_b, c2_W, c2_b):
    x = jnp.pad(obs[0], ((0, NPAD - N), (0, 0)))
    src = edge_index[0].astype(jnp.int32)
    dst = edge_index[1].astype(jnp.int32)
    npad_e = EP - E
    pad_i = jnp.arange(npad_e, dtype=jnp.int32)
    src_p = jnp.concatenate([src, (pad_i * 37) % N])
    dst_p = jnp.concatenate([dst, N + pad_i % TRASH])
    src_w = src_p.reshape(NW * NSUP, SUP, CHUNK)
    dst_w = dst_p.reshape(NW * NSUP, SUP, CHUNK)
    ones = jnp.ones((CHUNK,), jnp.float32)
    zeros = jnp.zeros((RPS,), jnp.float32)

    degp = _deg_kernel(dst_p.reshape(NW, NCHUNK, CHUNK), ones,
                       zeros).reshape(NC, NPAD)
    y1, dinv = _tc1(x, gcn_W1, degp)
    accp = _agg128(y1, src_w, dst_w).reshape(NC, NPAD, C)
    y2t = _tc2(accp, y1, dinv, gcn_b1, gcn_W2)
    acc2p = _agg2(y2t[0], y2t[1], src_w, dst_w).reshape(NC, 2, NPAD)
    am, sv = _tc3(acc2p, y2t, dinv, gcn_b2, lin_W, lin_b, a1_W, a1_b,
                  a2_W, a2_b, c1_W, c1_b, c2_W, c2_b)
    return am[:N], sv[:N, 0]


# speedup vs baseline: 32.0989x; 1.2914x over previous
import functools

import jax
import jax.numpy as jnp
from jax import lax
from jax.experimental import pallas as pl
from jax.experimental.pallas import tpu as pltpu
from jax.experimental.pallas import tpu_sc as plsc

N = 10000
E = 320000
C = 128
NC = 2
NS = 16
NW = NC * NS
CHUNK = 128
NCHUNK = 80
SUP = 10
NSUP = NCHUNK // SUP
EPW = NCHUNK * CHUNK
EP = NW * EPW
NPAD = 10240
RPS = NPAD // NS
TRASH = NPAD - N

_mesh = plsc.VectorSubcoreMesh(
    core_axis_name="c", subcore_axis_name="s", num_cores=NC, num_subcores=NS)


def _worker(c, s):
    return c * NS + s


@functools.partial(
    pl.kernel,
    out_type=jax.ShapeDtypeStruct((NC * NPAD,), jnp.float32),
    mesh=_mesh,
    scratch_types=[
        pltpu.VMEM((NCHUNK, CHUNK), jnp.int32),
        pltpu.VMEM((CHUNK,), jnp.float32),
        pltpu.VMEM_SHARED((NPAD,), jnp.float32),
        pltpu.SemaphoreType.DMA,
    ],
)
def _deg_kernel(dst_hbm, ones_hbm, zeros_hbm, out_hbm, idx_v, ones_v,
                deg_shared, ssem):
    c = lax.axis_index("c")
    s = lax.axis_index("s")
    w = _worker(c, s)
    pltpu.sync_copy(zeros_hbm, deg_shared.at[pl.ds(s * RPS, RPS)])
    pltpu.sync_copy(dst_hbm.at[w], idx_v)
    pltpu.sync_copy(ones_hbm, ones_v)
    plsc.subcore_barrier()

    def outer(g, carry):
        def fire(j, c2):
            pltpu.async_copy(ones_v, deg_shared.at[idx_v.at[g * 16 + j]],
                             ssem, add=True)
            return c2
        lax.fori_loop(0, 16, fire, 0)

        def drain(j, c2):
            pltpu.make_async_copy(ones_v, deg_shared.at[idx_v.at[0]],
                                  ssem).wait()
            return c2
        lax.fori_loop(0, 16, drain, 0)
        return carry
    lax.fori_loop(0, NCHUNK // 16, outer, 0)
    plsc.subcore_barrier()
    pltpu.sync_copy(deg_shared.at[pl.ds(s * RPS, RPS)],
                    out_hbm.at[pl.ds(c * NPAD + s * RPS, RPS)])


def _make_agg_kernel(width):
    @functools.partial(
        pl.kernel,
        out_type=jax.ShapeDtypeStruct((NC * NPAD, width), jnp.float32),
        mesh=_mesh,
        scratch_types=[
            pltpu.VMEM((SUP, CHUNK), jnp.int32),
            pltpu.VMEM((SUP, CHUNK), jnp.int32),
            pltpu.VMEM((2, CHUNK, width), jnp.float32),
            pltpu.VMEM_SHARED((NPAD, width), jnp.float32),
            pltpu.SemaphoreType.DMA,
            pltpu.SemaphoreType.DMA,
        ],
    )
    def agg(y_hbm, src_hbm, dst_hbm, out_hbm, src_v, dst_v, rows_v,
            acc_shared, gsem, ssem):
        c = lax.axis_index("c")
        s = lax.axis_index("s")
        w = _worker(c, s)
        pltpu.sync_copy(y_hbm.at[pl.ds(s * RPS, RPS)],
                        acc_shared.at[pl.ds(s * RPS, RPS)])
        plsc.subcore_barrier()

        def outer(g, carry):
            pltpu.sync_copy(src_hbm.at[w * NSUP + g], src_v)
            pltpu.sync_copy(dst_hbm.at[w * NSUP + g], dst_v)
            pltpu.async_copy(y_hbm.at[src_v.at[0]], rows_v.at[0], gsem)

            def body(j, carry2):
                nxt = j + 1
                @pl.when(nxt < SUP)
                def _start():
                    @pl.when(j >= 1)
                    def _free():
                        pltpu.make_async_copy(
                            rows_v.at[lax.rem(nxt, 2)],
                            acc_shared.at[dst_v.at[0]], ssem).wait()
                    pltpu.async_copy(y_hbm.at[src_v.at[nxt]],
                                     rows_v.at[lax.rem(nxt, 2)], gsem)
                pltpu.make_async_copy(y_hbm.at[src_v.at[j]],
                                      rows_v.at[lax.rem(j, 2)], gsem).wait()
                pltpu.async_copy(rows_v.at[lax.rem(j, 2)],
                                 acc_shared.at[dst_v.at[j]], ssem, add=True)
                return carry2
            lax.fori_loop(0, SUP, body, 0)
            pltpu.make_async_copy(rows_v.at[0],
                                  acc_shared.at[dst_v.at[0]], ssem).wait()
            pltpu.make_async_copy(rows_v.at[1],
                                  acc_shared.at[dst_v.at[0]], ssem).wait()
            return carry
        lax.fori_loop(0, NSUP, outer, 0)
        plsc.subcore_barrier()
        pltpu.sync_copy(acc_shared.at[pl.ds(s * RPS, RPS)],
                        out_hbm.at[pl.ds(c * NPAD + s * RPS, RPS)])
    return agg


_agg128 = _make_agg_kernel(C)


@functools.partial(
    pl.kernel,
    out_type=jax.ShapeDtypeStruct((NC * 2 * NPAD,), jnp.float32),
    mesh=_mesh,
    scratch_types=[
        pltpu.VMEM((SUP, CHUNK), jnp.int32),
        pltpu.VMEM((SUP, CHUNK), jnp.int32),
        pltpu.VMEM((2, CHUNK), jnp.float32),
        pltpu.VMEM((2, CHUNK), jnp.float32),
        pltpu.VMEM_SHARED((NPAD,), jnp.float32),
        pltpu.VMEM_SHARED((NPAD,), jnp.float32),
        pltpu.SemaphoreType.DMA,
        pltpu.SemaphoreType.DMA,
    ],
)
def _agg2(y0_hbm, y1_hbm, src_hbm, dst_hbm, out_hbm, src_v, dst_v,
          v0, v1, acc0_sh, acc1_sh, gsem, ssem):
    c = lax.axis_index("c")
    s = lax.axis_index("s")
    w = _worker(c, s)
    pltpu.sync_copy(y0_hbm.at[pl.ds(s * RPS, RPS)],
                    acc0_sh.at[pl.ds(s * RPS, RPS)])
    pltpu.sync_copy(y1_hbm.at[pl.ds(s * RPS, RPS)],
                    acc1_sh.at[pl.ds(s * RPS, RPS)])
    plsc.subcore_barrier()

    def outer(g, carry):
        pltpu.sync_copy(src_hbm.at[w * NSUP + g], src_v)
        pltpu.sync_copy(dst_hbm.at[w * NSUP + g], dst_v)
        pltpu.async_copy(y0_hbm.at[src_v.at[0]], v0.at[0], gsem)
        pltpu.async_copy(y1_hbm.at[src_v.at[0]], v1.at[0], gsem)

        def body(j, carry2):
            nxt = j + 1
            @pl.when(nxt < SUP)
            def _start():
                @pl.when(j >= 1)
                def _free():
                    pltpu.make_async_copy(v0.at[lax.rem(nxt, 2)],
                                          acc0_sh.at[dst_v.at[0]],
                                          ssem).wait()
                    pltpu.make_async_copy(v1.at[lax.rem(nxt, 2)],
                                          acc1_sh.at[dst_v.at[0]],
                                          ssem).wait()
                pltpu.async_copy(y0_hbm.at[src_v.at[nxt]],
                                 v0.at[lax.rem(nxt, 2)], gsem)
                pltpu.async_copy(y1_hbm.at[src_v.at[nxt]],
                                 v1.at[lax.rem(nxt, 2)], gsem)
            pltpu.make_async_copy(y0_hbm.at[src_v.at[j]],
                                  v0.at[lax.rem(j, 2)], gsem).wait()
            pltpu.make_async_copy(y1_hbm.at[src_v.at[j]],
                                  v1.at[lax.rem(j, 2)], gsem).wait()
            pltpu.async_copy(v0.at[lax.rem(j, 2)],
                             acc0_sh.at[dst_v.at[j]], ssem, add=True)
            pltpu.async_copy(v1.at[lax.rem(j, 2)],
                             acc1_sh.at[dst_v.at[j]], ssem, add=True)
            return carry2
        lax.fori_loop(0, SUP, body, 0)
        pltpu.make_async_copy(v0.at[0], acc0_sh.at[dst_v.at[0]], ssem).wait()
        pltpu.make_async_copy(v1.at[0], acc1_sh.at[dst_v.at[0]], ssem).wait()
        pltpu.make_async_copy(v0.at[1], acc0_sh.at[dst_v.at[0]], ssem).wait()
        pltpu.make_async_copy(v1.at[1], acc1_sh.at[dst_v.at[0]], ssem).wait()
        return carry
    lax.fori_loop(0, NSUP, outer, 0)
    plsc.subcore_barrier()
    pltpu.sync_copy(acc0_sh.at[pl.ds(s * RPS, RPS)],
                    out_hbm.at[pl.ds(c * 2 * NPAD + s * RPS, RPS)])
    pltpu.sync_copy(acc1_sh.at[pl.ds(s * RPS, RPS)],
                    out_hbm.at[pl.ds(c * 2 * NPAD + NPAD + s * RPS, RPS)])


_BLK = 512
_GRID = NPAD // _BLK


def _tc1_body(x_ref, w1_ref, degp_ref, y1_ref, dinv_ref):
    deg = degp_ref[0, :] + degp_ref[1, :] + 1.0
    dinv = 1.0 / jnp.sqrt(deg)
    h = jnp.dot(x_ref[...], w1_ref[...], preferred_element_type=jnp.float32)
    y1_ref[...] = h * dinv[:, None]
    dinv_ref[...] = dinv


def _tc1(x, w1, degp):
    return pl.pallas_call(
        _tc1_body,
        grid=(_GRID,),
        in_specs=[
            pl.BlockSpec((_BLK, C), lambda i: (i, 0)),
            pl.BlockSpec((C, C), lambda i: (0, 0)),
            pl.BlockSpec((NC, _BLK), lambda i: (0, i)),
        ],
        out_specs=[
            pl.BlockSpec((_BLK, C), lambda i: (i, 0)),
            pl.BlockSpec((_BLK,), lambda i: (i,)),
        ],
        out_shape=[
            jax.ShapeDtypeStruct((NPAD, C), jnp.float32),
            jax.ShapeDtypeStruct((NPAD,), jnp.float32),
        ],
    )(x, w1, degp)


def _tc2_body(accp_ref, y1_ref, dinv_ref, b1_ref, w2_ref, y2t_ref):
    agg = accp_ref[0] + accp_ref[1] - y1_ref[...]
    dinv = dinv_ref[...]
    out1 = jax.nn.relu(agg * dinv[:, None] + b1_ref[...][None, :])
    h2t = lax.dot_general(w2_ref[...], out1, (((0,), (1,)), ((), ())),
                          preferred_element_type=jnp.float32)
    y2t_ref[...] = h2t * dinv[None, :]


def _tc2(accp, y1, dinv, b1, w2):
    return pl.pallas_call(
        _tc2_body,
        grid=(_GRID,),
        in_specs=[
            pl.BlockSpec((NC, _BLK, C), lambda i: (0, i, 0)),
            pl.BlockSpec((_BLK, C), lambda i: (i, 0)),
            pl.BlockSpec((_BLK,), lambda i: (i,)),
            pl.BlockSpec((C,), lambda i: (0,)),
            pl.BlockSpec((C, 2), lambda i: (0, 0)),
        ],
        out_specs=pl.BlockSpec((2, _BLK), lambda i: (0, i)),
        out_shape=jax.ShapeDtypeStruct((2, NPAD), jnp.float32),
    )(accp, y1, dinv, b1, w2)


def _tc3_body(acc2p_ref, y2_ref, dinv_ref, b2_ref, linw_ref, linb_ref,
              a1w_ref, a1b_ref, a2w_ref, a2b_ref, c1w_ref, c1b_ref,
              c2w_ref, c2b_ref, am_ref, sv_ref):
    agg = (acc2p_ref[0] + acc2p_ref[1] - y2_ref[...]).T
    dinv = dinv_ref[...]
    out2 = jax.nn.relu(agg * dinv[:, None] + b2_ref[...][None, :])
    feats = jnp.dot(out2, linw_ref[...],
                    preferred_element_type=jnp.float32) + linb_ref[...][None, :]
    a = jax.nn.relu(jnp.dot(feats, a1w_ref[...],
                            preferred_element_type=jnp.float32)
                    + a1b_ref[...][None, :])
    logits = jnp.dot(a, a2w_ref[...],
                     preferred_element_type=jnp.float32) + a2b_ref[...][None, :]
    m = jnp.max(logits, axis=-1, keepdims=True)
    ex = jnp.exp(logits - m)
    am_ref[...] = ex / jnp.sum(ex, axis=-1, keepdims=True)
    cv = jax.nn.relu(jnp.dot(feats, c1w_ref[...],
                             preferred_element_type=jnp.float32)
                     + c1b_ref[...][None, :])
    sv_ref[...] = jnp.dot(cv, c2w_ref[...],
                          preferred_element_type=jnp.float32) + c2b_ref[...][None, :]


def _tc3(acc2p, y2, dinv, b2, lin_w, lin_b, a1w, a1b, a2w, a2b, c1w, c1b,
         c2w, c2b):
    return pl.pallas_call(
        _tc3_body,
        grid=(_GRID,),
        in_specs=[
            pl.BlockSpec((NC, 2, _BLK), lambda i: (0, 0, i)),
            pl.BlockSpec((2, _BLK), lambda i: (0, i)),
            pl.BlockSpec((_BLK,), lambda i: (i,)),
            pl.BlockSpec((2,), lambda i: (0,)),
            pl.BlockSpec((2, 2), lambda i: (0, 0)),
            pl.BlockSpec((2,), lambda i: (0,)),
            pl.BlockSpec((2, C), lambda i: (0, 0)),
            pl.BlockSpec((C,), lambda i: (0,)),
            pl.BlockSpec((C, 10), lambda i: (0, 0)),
            pl.BlockSpec((10,), lambda i: (0,)),
            pl.BlockSpec((2, C), lambda i: (0, 0)),
            pl.BlockSpec((C,), lambda i: (0,)),
            pl.BlockSpec((C, 1), lambda i: (0, 0)),
            pl.BlockSpec((1,), lambda i: (0,)),
        ],
        out_specs=[
            pl.BlockSpec((_BLK, 10), lambda i: (i, 0)),
            pl.BlockSpec((_BLK, 1), lambda i: (i, 0)),
        ],
        out_shape=[
            jax.ShapeDtypeStruct((NPAD, 10), jnp.float32),
            jax.ShapeDtypeStruct((NPAD, 1), jnp.float32),
        ],
    )(acc2p, y2, dinv, b2, lin_w, lin_b, a1w, a1b, a2w, a2b, c1w, c1b,
      c2w, c2b)


def kernel(obs, edge_index, gcn_W1, gcn_b1, gcn_W2, gcn_b2, lin_W, lin_b,
           a1_W, a1_b, a2_W, a2_b, c1_W, c1_b, c2_W, c2_b):
    x = jnp.pad(obs[0], ((0, NPAD - N), (0, 0)))
    src = edge_index[0].astype(jnp.int32)
    dst = edge_index[1].astype(jnp.int32)
    npad_e = EP - E
    pad_i = jnp.arange(npad_e, dtype=jnp.int32)
    src_p = jnp.concatenate([src, (pad_i * 37) % N])
    dst_p = jnp.concatenate([dst, N + pad_i % TRASH])
    src_w = src_p.reshape(NW * NSUP, SUP, CHUNK)
    dst_w = dst_p.reshape(NW * NSUP, SUP, CHUNK)
    ones = jnp.ones((CHUNK,), jnp.float32)
    zeros = jnp.zeros((RPS,), jnp.float32)

    degp = _deg_kernel(dst_p.reshape(NW, NCHUNK, CHUNK), ones,
                       zeros).reshape(NC, NPAD)
    y1, dinv = _tc1(x, gcn_W1, degp)
    accp = _agg128(y1, src_w, dst_w).reshape(NC, NPAD, C)
    y2t = _tc2(accp, y1, dinv, gcn_b1, gcn_W2)
    acc2p = _agg2(y2t[0], y2t[1], src_w, dst_w).reshape(NC, 2, NPAD)
    am, sv = _tc3(acc2p, y2t, dinv, gcn_b2, lin_W, lin_b, a1_W, a1_b,
                  a2_W, a2_b, c1_W, c1_b, c2_W, c2_b)
    return am[:N], sv[:N, 0]


# speedup vs baseline: 34.0440x; 1.0606x over previous
import functools

import jax
import jax.numpy as jnp
from jax import lax
from jax.experimental import pallas as pl
from jax.experimental.pallas import tpu as pltpu
from jax.experimental.pallas import tpu_sc as plsc

N = 10000
E = 320000
C = 128
NC = 2
NS = 16
NW = NC * NS
CHUNK = 128
NCHUNK = 80
SUP = 10
NSUP = NCHUNK // SUP
EPW = NCHUNK * CHUNK
EP = NW * EPW
NPAD = 10240
RPS = NPAD // NS
TRASH = NPAD - N

_mesh = plsc.VectorSubcoreMesh(
    core_axis_name="c", subcore_axis_name="s", num_cores=NC, num_subcores=NS)


def _worker(c, s):
    return c * NS + s


@functools.partial(
    pl.kernel,
    out_type=jax.ShapeDtypeStruct((NC * NPAD,), jnp.float32),
    mesh=_mesh,
    scratch_types=[
        pltpu.VMEM((NCHUNK, CHUNK), jnp.int32),
        pltpu.VMEM((CHUNK,), jnp.float32),
        pltpu.VMEM_SHARED((NPAD,), jnp.float32),
        pltpu.SemaphoreType.DMA,
    ],
)
def _deg_kernel(dst_hbm, ones_hbm, zeros_hbm, out_hbm, idx_v, ones_v,
                deg_shared, ssem):
    c = lax.axis_index("c")
    s = lax.axis_index("s")
    w = _worker(c, s)
    pltpu.sync_copy(zeros_hbm, deg_shared.at[pl.ds(s * RPS, RPS)])
    pltpu.sync_copy(dst_hbm.at[w], idx_v)
    pltpu.sync_copy(ones_hbm, ones_v)
    plsc.subcore_barrier()

    def outer(g, carry):
        def fire(j, c2):
            pltpu.async_copy(ones_v, deg_shared.at[idx_v.at[g * 16 + j]],
                             ssem, add=True)
            return c2
        lax.fori_loop(0, 16, fire, 0)

        def drain(j, c2):
            pltpu.make_async_copy(ones_v, deg_shared.at[idx_v.at[0]],
                                  ssem).wait()
            return c2
        lax.fori_loop(0, 16, drain, 0)
        return carry
    lax.fori_loop(0, NCHUNK // 16, outer, 0)
    plsc.subcore_barrier()
    pltpu.sync_copy(deg_shared.at[pl.ds(s * RPS, RPS)],
                    out_hbm.at[pl.ds(c * NPAD + s * RPS, RPS)])


def _make_agg_kernel(width):
    @functools.partial(
        pl.kernel,
        out_type=jax.ShapeDtypeStruct((NC * NPAD, width), jnp.float32),
        mesh=_mesh,
        scratch_types=[
            pltpu.VMEM((SUP, CHUNK), jnp.int32),
            pltpu.VMEM((SUP, CHUNK), jnp.int32),
            pltpu.VMEM((2, CHUNK, width), jnp.float32),
            pltpu.VMEM_SHARED((NPAD, width), jnp.float32),
            pltpu.SemaphoreType.DMA,
            pltpu.SemaphoreType.DMA,
        ],
    )
    def agg(y_hbm, src_hbm, dst_hbm, out_hbm, src_v, dst_v, rows_v,
            acc_shared, gsem, ssem):
        c = lax.axis_index("c")
        s = lax.axis_index("s")
        w = _worker(c, s)
        pltpu.sync_copy(y_hbm.at[pl.ds(s * RPS, RPS)],
                        acc_shared.at[pl.ds(s * RPS, RPS)])
        plsc.subcore_barrier()

        def outer(g, carry):
            pltpu.sync_copy(src_hbm.at[w * NSUP + g], src_v)
            pltpu.sync_copy(dst_hbm.at[w * NSUP + g], dst_v)
            pltpu.async_copy(y_hbm.at[src_v.at[0]], rows_v.at[0], gsem)

            def body(j, carry2):
                nxt = j + 1
                @pl.when(nxt < SUP)
                def _start():
                    @pl.when(j >= 1)
                    def _free():
                        pltpu.make_async_copy(
                            rows_v.at[lax.rem(nxt, 2)],
                            acc_shared.at[dst_v.at[0]], ssem).wait()
                    pltpu.async_copy(y_hbm.at[src_v.at[nxt]],
                                     rows_v.at[lax.rem(nxt, 2)], gsem)
                pltpu.make_async_copy(y_hbm.at[src_v.at[j]],
                                      rows_v.at[lax.rem(j, 2)], gsem).wait()
                pltpu.async_copy(rows_v.at[lax.rem(j, 2)],
                                 acc_shared.at[dst_v.at[j]], ssem, add=True)
                return carry2
            lax.fori_loop(0, SUP, body, 0)
            pltpu.make_async_copy(rows_v.at[0],
                                  acc_shared.at[dst_v.at[0]], ssem).wait()
            pltpu.make_async_copy(rows_v.at[1],
                                  acc_shared.at[dst_v.at[0]], ssem).wait()
            return carry
        lax.fori_loop(0, NSUP, outer, 0)
        plsc.subcore_barrier()
        pltpu.sync_copy(acc_shared.at[pl.ds(s * RPS, RPS)],
                        out_hbm.at[pl.ds(c * NPAD + s * RPS, RPS)])
    return agg


_agg128 = _make_agg_kernel(C)


@functools.partial(
    pl.kernel,
    out_type=jax.ShapeDtypeStruct((NC * 2 * NPAD,), jnp.float32),
    mesh=_mesh,
    scratch_types=[
        pltpu.VMEM((SUP, CHUNK), jnp.int32),
        pltpu.VMEM((SUP, CHUNK), jnp.int32),
        pltpu.VMEM((2, CHUNK), jnp.float32),
        pltpu.VMEM((2, CHUNK), jnp.float32),
        pltpu.VMEM_SHARED((NPAD,), jnp.float32),
        pltpu.VMEM_SHARED((NPAD,), jnp.float32),
        pltpu.SemaphoreType.DMA,
        pltpu.SemaphoreType.DMA,
    ],
)
def _agg2(y0_hbm, y1_hbm, src_hbm, dst_hbm, out_hbm, src_v, dst_v,
          v0, v1, acc0_sh, acc1_sh, gsem, ssem):
    c = lax.axis_index("c")
    s = lax.axis_index("s")
    w = _worker(c, s)
    pltpu.sync_copy(y0_hbm.at[pl.ds(s * RPS, RPS)],
                    acc0_sh.at[pl.ds(s * RPS, RPS)])
    pltpu.sync_copy(y1_hbm.at[pl.ds(s * RPS, RPS)],
                    acc1_sh.at[pl.ds(s * RPS, RPS)])
    plsc.subcore_barrier()

    def outer(g, carry):
        pltpu.sync_copy(src_hbm.at[w * NSUP + g], src_v)
        pltpu.sync_copy(dst_hbm.at[w * NSUP + g], dst_v)
        pltpu.async_copy(y0_hbm.at[src_v.at[0]], v0.at[0], gsem)
        pltpu.async_copy(y1_hbm.at[src_v.at[0]], v1.at[0], gsem)

        def body(j, carry2):
            nxt = j + 1
            @pl.when(nxt < SUP)
            def _start():
                @pl.when(j >= 1)
                def _free():
                    pltpu.make_async_copy(v0.at[lax.rem(nxt, 2)],
                                          acc0_sh.at[dst_v.at[0]],
                                          ssem).wait()
                    pltpu.make_async_copy(v1.at[lax.rem(nxt, 2)],
                                          acc1_sh.at[dst_v.at[0]],
                                          ssem).wait()
                pltpu.async_copy(y0_hbm.at[src_v.at[nxt]],
                                 v0.at[lax.rem(nxt, 2)], gsem)
                pltpu.async_copy(y1_hbm.at[src_v.at[nxt]],
                                 v1.at[lax.rem(nxt, 2)], gsem)
            pltpu.make_async_copy(y0_hbm.at[src_v.at[j]],
                                  v0.at[lax.rem(j, 2)], gsem).wait()
            pltpu.make_async_copy(y1_hbm.at[src_v.at[j]],
                                  v1.at[lax.rem(j, 2)], gsem).wait()
            pltpu.async_copy(v0.at[lax.rem(j, 2)],
                             acc0_sh.at[dst_v.at[j]], ssem, add=True)
            pltpu.async_copy(v1.at[lax.rem(j, 2)],
                             acc1_sh.at[dst_v.at[j]], ssem, add=True)
            return carry2
        lax.fori_loop(0, SUP, body, 0)
        pltpu.make_async_copy(v0.at[0], acc0_sh.at[dst_v.at[0]], ssem).wait()
        pltpu.make_async_copy(v1.at[0], acc1_sh.at[dst_v.at[0]], ssem).wait()
        pltpu.make_async_copy(v0.at[1], acc0_sh.at[dst_v.at[0]], ssem).wait()
        pltpu.make_async_copy(v1.at[1], acc1_sh.at[dst_v.at[0]], ssem).wait()
        return carry
    lax.fori_loop(0, NSUP, outer, 0)
    plsc.subcore_barrier()
    pltpu.sync_copy(acc0_sh.at[pl.ds(s * RPS, RPS)],
                    out_hbm.at[pl.ds(c * 2 * NPAD + s * RPS, RPS)])
    pltpu.sync_copy(acc1_sh.at[pl.ds(s * RPS, RPS)],
                    out_hbm.at[pl.ds(c * 2 * NPAD + NPAD + s * RPS, RPS)])


_BLK = 1024
_GRID = NPAD // _BLK


def _tc1_body(x_ref, w1_ref, degp_ref, y1_ref, dinv_ref):
    deg = degp_ref[0, :] + degp_ref[1, :] + 1.0
    dinv = 1.0 / jnp.sqrt(deg)
    h = jnp.dot(x_ref[...], w1_ref[...], preferred_element_type=jnp.float32)
    y1_ref[...] = h * dinv[:, None]
    dinv_ref[...] = dinv


def _tc1(x, w1, degp):
    return pl.pallas_call(
        _tc1_body,
        grid=(_GRID,),
        in_specs=[
            pl.BlockSpec((_BLK, C), lambda i: (i, 0)),
            pl.BlockSpec((C, C), lambda i: (0, 0)),
            pl.BlockSpec((NC, _BLK), lambda i: (0, i)),
        ],
        out_specs=[
            pl.BlockSpec((_BLK, C), lambda i: (i, 0)),
            pl.BlockSpec((_BLK,), lambda i: (i,)),
        ],
        out_shape=[
            jax.ShapeDtypeStruct((NPAD, C), jnp.float32),
            jax.ShapeDtypeStruct((NPAD,), jnp.float32),
        ],
    )(x, w1, degp)


def _tc2_body(accp_ref, y1_ref, dinv_ref, b1_ref, w2_ref, y2t_ref):
    agg = accp_ref[0] + accp_ref[1] - y1_ref[...]
    dinv = dinv_ref[...]
    out1 = jax.nn.relu(agg * dinv[:, None] + b1_ref[...][None, :])
    h2t = lax.dot_general(w2_ref[...], out1, (((0,), (1,)), ((), ())),
                          preferred_element_type=jnp.float32)
    y2t_ref[...] = h2t * dinv[None, :]


def _tc2(accp, y1, dinv, b1, w2):
    return pl.pallas_call(
        _tc2_body,
        grid=(_GRID,),
        in_specs=[
            pl.BlockSpec((NC, _BLK, C), lambda i: (0, i, 0)),
            pl.BlockSpec((_BLK, C), lambda i: (i, 0)),
            pl.BlockSpec((_BLK,), lambda i: (i,)),
            pl.BlockSpec((C,), lambda i: (0,)),
            pl.BlockSpec((C, 2), lambda i: (0, 0)),
        ],
        out_specs=pl.BlockSpec((2, _BLK), lambda i: (0, i)),
        out_shape=jax.ShapeDtypeStruct((2, NPAD), jnp.float32),
    )(accp, y1, dinv, b1, w2)


def _tc3_body(acc2p_ref, y2_ref, dinv_ref, b2_ref, linw_ref, linb_ref,
              a1w_ref, a1b_ref, a2w_ref, a2b_ref, c1w_ref, c1b_ref,
              c2w_ref, c2b_ref, am_ref, sv_ref):
    agg = (acc2p_ref[0] + acc2p_ref[1] - y2_ref[...]).T
    dinv = dinv_ref[...]
    out2 = jax.nn.relu(agg * dinv[:, None] + b2_ref[...][None, :])
    feats = jnp.dot(out2, linw_ref[...],
                    preferred_element_type=jnp.float32) + linb_ref[...][None, :]
    a = jax.nn.relu(jnp.dot(feats, a1w_ref[...],
                            preferred_element_type=jnp.float32)
                    + a1b_ref[...][None, :])
    logits = jnp.dot(a, a2w_ref[...],
                     preferred_element_type=jnp.float32) + a2b_ref[...][None, :]
    m = jnp.max(logits, axis=-1, keepdims=True)
    ex = jnp.exp(logits - m)
    am_ref[...] = ex / jnp.sum(ex, axis=-1, keepdims=True)
    cv = jax.nn.relu(jnp.dot(feats, c1w_ref[...],
                             preferred_element_type=jnp.float32)
                     + c1b_ref[...][None, :])
    sv_ref[...] = jnp.dot(cv, c2w_ref[...],
                          preferred_element_type=jnp.float32) + c2b_ref[...][None, :]


def _tc3(acc2p, y2, dinv, b2, lin_w, lin_b, a1w, a1b, a2w, a2b, c1w, c1b,
         c2w, c2b):
    return pl.pallas_call(
        _tc3_body,
        grid=(_GRID,),
        in_specs=[
            pl.BlockSpec((NC, 2, _BLK), lambda i: (0, 0, i)),
            pl.BlockSpec((2, _BLK), lambda i: (0, i)),
            pl.BlockSpec((_BLK,), lambda i: (i,)),
            pl.BlockSpec((2,), lambda i: (0,)),
            pl.BlockSpec((2, 2), lambda i: (0, 0)),
            pl.BlockSpec((2,), lambda i: (0,)),
            pl.BlockSpec((2, C), lambda i: (0, 0)),
            pl.BlockSpec((C,), lambda i: (0,)),
            pl.BlockSpec((C, 10), lambda i: (0, 0)),
            pl.BlockSpec((10,), lambda i: (0,)),
            pl.BlockSpec((2, C), lambda i: (0, 0)),
            pl.BlockSpec((C,), lambda i: (0,)),
            pl.BlockSpec((C, 1), lambda i: (0, 0)),
            pl.BlockSpec((1,), lambda i: (0,)),
        ],
        out_specs=[
            pl.BlockSpec((_BLK, 10), lambda i: (i, 0)),
            pl.BlockSpec((_BLK, 1), lambda i: (i, 0)),
        ],
        out_shape=[
            jax.ShapeDtypeStruct((NPAD, 10), jnp.float32),
            jax.ShapeDtypeStruct((NPAD, 1), jnp.float32),
        ],
    )(acc2p, y2, dinv, b2, lin_w, lin_b, a1w, a1b, a2w, a2b, c1w, c1b,
      c2w, c2b)


def kernel(obs, edge_index, gcn_W1, gcn_b1, gcn_W2, gcn_b2, lin_W, lin_b,
           a1_W, a1_b, a2_W, a2_b, c1_W, c1_b, c2_W, c2_b):
    x = jnp.pad(obs[0], ((0, NPAD - N), (0, 0)))
    src = edge_index[0].astype(jnp.int32)
    dst = edge_index[1].astype(jnp.int32)
    npad_e = EP - E
    pad_i = jnp.arange(npad_e, dtype=jnp.int32)
    src_p = jnp.concatenate([src, (pad_i * 37) % N])
    dst_p = jnp.concatenate([dst, N + pad_i % TRASH])
    src_w = src_p.reshape(NW * NSUP, SUP, CHUNK)
    dst_w = dst_p.reshape(NW * NSUP, SUP, CHUNK)
    ones = jnp.ones((CHUNK,), jnp.float32)
    zeros = jnp.zeros((RPS,), jnp.float32)

    degp = _deg_kernel(dst_p.reshape(NW, NCHUNK, CHUNK), ones,
                       zeros).reshape(NC, NPAD)
    y1, dinv = _tc1(x, gcn_W1, degp)
    accp = _agg128(y1, src_w, dst_w).reshape(NC, NPAD, C)
    y2t = _tc2(accp, y1, dinv, gcn_b1, gcn_W2)
    acc2p = _agg2(y2t[0], y2t[1], src_w, dst_w).reshape(NC, 2, NPAD)
    am, sv = _tc3(acc2p, y2t, dinv, gcn_b2, lin_W, lin_b, a1_W, a1_b,
                  a2_W, a2_b, c1_W, c1_b, c2_W, c2_b)
    return am[:N], sv[:N, 0]


# speedup vs baseline: 40.2594x; 1.1826x over previous
import functools

import jax
import jax.numpy as jnp
from jax import lax
from jax.experimental import pallas as pl
from jax.experimental.pallas import tpu as pltpu
from jax.experimental.pallas import tpu_sc as plsc

N = 10000
E = 320000
C = 128
NC = 2
NS = 16
NW = NC * NS
CHUNK = 128
NCHUNK = 80
SUP = 10
NSUP = NCHUNK // SUP
EPW = NCHUNK * CHUNK
EP = NW * EPW
NPAD = 10240
RPS = NPAD // NS
TRASH = NPAD - N

_mesh = plsc.VectorSubcoreMesh(
    core_axis_name="c", subcore_axis_name="s", num_cores=NC, num_subcores=NS)


def _worker(c, s):
    return c * NS + s


@functools.partial(
    pl.kernel,
    out_type=jax.ShapeDtypeStruct((NC * NPAD,), jnp.float32),
    mesh=_mesh,
    scratch_types=[
        pltpu.VMEM((NCHUNK, CHUNK), jnp.int32),
        pltpu.VMEM((CHUNK,), jnp.float32),
        pltpu.VMEM_SHARED((NPAD,), jnp.float32),
        pltpu.SemaphoreType.DMA,
    ],
)
def _deg_kernel(dst_hbm, ones_hbm, zeros_hbm, out_hbm, idx_v, ones_v,
                deg_shared, ssem):
    c = lax.axis_index("c")
    s = lax.axis_index("s")
    w = _worker(c, s)
    pltpu.sync_copy(zeros_hbm, deg_shared.at[pl.ds(s * RPS, RPS)])
    pltpu.sync_copy(dst_hbm.at[w], idx_v)
    pltpu.sync_copy(ones_hbm, ones_v)
    plsc.subcore_barrier()

    def outer(g, carry):
        def fire(j, c2):
            pltpu.async_copy(ones_v, deg_shared.at[idx_v.at[g * 16 + j]],
                             ssem, add=True)
            return c2
        lax.fori_loop(0, 16, fire, 0)

        def drain(j, c2):
            pltpu.make_async_copy(ones_v, deg_shared.at[idx_v.at[0]],
                                  ssem).wait()
            return c2
        lax.fori_loop(0, 16, drain, 0)
        return carry
    lax.fori_loop(0, NCHUNK // 16, outer, 0)
    plsc.subcore_barrier()
    pltpu.sync_copy(deg_shared.at[pl.ds(s * RPS, RPS)],
                    out_hbm.at[pl.ds(c * NPAD + s * RPS, RPS)])


def _make_agg_kernel(width):
    @functools.partial(
        pl.kernel,
        out_type=jax.ShapeDtypeStruct((NC * NPAD, width), jnp.float32),
        mesh=_mesh,
        scratch_types=[
            pltpu.VMEM((SUP, CHUNK), jnp.int32),
            pltpu.VMEM((SUP, CHUNK), jnp.int32),
            pltpu.VMEM((2, CHUNK, width), jnp.float32),
            pltpu.VMEM_SHARED((NPAD, width), jnp.float32),
            pltpu.SemaphoreType.DMA,
            pltpu.SemaphoreType.DMA,
        ],
    )
    def agg(y_hbm, src_hbm, dst_hbm, out_hbm, src_v, dst_v, rows_v,
            acc_shared, gsem, ssem):
        c = lax.axis_index("c")
        s = lax.axis_index("s")
        w = _worker(c, s)
        pltpu.sync_copy(y_hbm.at[pl.ds(s * RPS, RPS)],
                        acc_shared.at[pl.ds(s * RPS, RPS)])
        plsc.subcore_barrier()

        def outer(g, carry):
            pltpu.sync_copy(src_hbm.at[w * NSUP + g], src_v)
            pltpu.sync_copy(dst_hbm.at[w * NSUP + g], dst_v)
            pltpu.async_copy(y_hbm.at[src_v.at[0]], rows_v.at[0], gsem)

            def body(j, carry2):
                nxt = j + 1
                @pl.when(nxt < SUP)
                def _start():
                    @pl.when(j >= 1)
                    def _free():
                        pltpu.make_async_copy(
                            rows_v.at[lax.rem(nxt, 2)],
                            acc_shared.at[dst_v.at[0]], ssem).wait()
                    pltpu.async_copy(y_hbm.at[src_v.at[nxt]],
                                     rows_v.at[lax.rem(nxt, 2)], gsem)
                pltpu.make_async_copy(y_hbm.at[src_v.at[j]],
                                      rows_v.at[lax.rem(j, 2)], gsem).wait()
                pltpu.async_copy(rows_v.at[lax.rem(j, 2)],
                                 acc_shared.at[dst_v.at[j]], ssem, add=True)
                return carry2
            lax.fori_loop(0, SUP, body, 0)
            pltpu.make_async_copy(rows_v.at[0],
                                  acc_shared.at[dst_v.at[0]], ssem).wait()
            pltpu.make_async_copy(rows_v.at[1],
                                  acc_shared.at[dst_v.at[0]], ssem).wait()
            return carry
        lax.fori_loop(0, NSUP, outer, 0)
        plsc.subcore_barrier()
        pltpu.sync_copy(acc_shared.at[pl.ds(s * RPS, RPS)],
                        out_hbm.at[pl.ds(c * NPAD + s * RPS, RPS)])
    return agg


_agg128 = _make_agg_kernel(C)


@functools.partial(
    pl.kernel,
    out_type=jax.ShapeDtypeStruct((NC * 2 * NPAD,), jnp.float32),
    mesh=_mesh,
    scratch_types=[
        pltpu.VMEM((SUP * CHUNK,), jnp.int32),
        pltpu.VMEM((SUP, CHUNK), jnp.int32),
        pltpu.VMEM((NPAD,), jnp.float32),
        pltpu.VMEM((NPAD,), jnp.float32),
        pltpu.VMEM((2, CHUNK), jnp.float32),
        pltpu.VMEM((2, CHUNK), jnp.float32),
        pltpu.VMEM_SHARED((NPAD,), jnp.float32),
        pltpu.VMEM_SHARED((NPAD,), jnp.float32),
        pltpu.SemaphoreType.DMA,
    ],
    compiler_params=pltpu.CompilerParams(needs_layout_passes=False),
)
def _agg2(yt_hbm, src_hbm, dst_hbm, out_hbm, src_v, dst_v,
          ytab0_v, ytab1_v, upd0_v, upd1_v, acc0_sh, acc1_sh, ssem):
    c = lax.axis_index("c")
    s = lax.axis_index("s")
    w = _worker(c, s)
    pltpu.sync_copy(yt_hbm.at[0].at[pl.ds(s * RPS, RPS)],
                    acc0_sh.at[pl.ds(s * RPS, RPS)])
    pltpu.sync_copy(yt_hbm.at[1].at[pl.ds(s * RPS, RPS)],
                    acc1_sh.at[pl.ds(s * RPS, RPS)])
    pltpu.sync_copy(yt_hbm.at[0], ytab0_v)
    pltpu.sync_copy(yt_hbm.at[1], ytab1_v)
    plsc.subcore_barrier()

    def outer(g, carry):
        pltpu.sync_copy(src_hbm.at[w * NSUP + g], src_v)
        pltpu.sync_copy(dst_hbm.at[w * NSUP + g], dst_v)

        def body(j, carry2):
            b = lax.rem(j, 2)
            @pl.when(j >= 2)
            def _free():
                pltpu.make_async_copy(upd0_v.at[0],
                                      acc0_sh.at[dst_v.at[0]], ssem).wait()
                pltpu.make_async_copy(upd1_v.at[0],
                                      acc1_sh.at[dst_v.at[0]], ssem).wait()
            lane = jnp.arange(16, dtype=jnp.int32)
            bvec = lane * 0 + b
            for k in range(8):
                src16 = plsc.load_gather(src_v, [j * CHUNK + k * 16 + lane])
                v0 = plsc.load_gather(ytab0_v, [src16])
                v1 = plsc.load_gather(ytab1_v, [src16])
                row = k * 16 + lane
                plsc.store_scatter(upd0_v, [bvec, row], v0)
                plsc.store_scatter(upd1_v, [bvec, row], v1)
            pltpu.async_copy(upd0_v.at[b], acc0_sh.at[dst_v.at[j]], ssem,
                             add=True)
            pltpu.async_copy(upd1_v.at[b], acc1_sh.at[dst_v.at[j]], ssem,
                             add=True)
            return carry2
        lax.fori_loop(0, SUP, body, 0)
        pltpu.make_async_copy(upd0_v.at[0], acc0_sh.at[dst_v.at[0]],
                              ssem).wait()
        pltpu.make_async_copy(upd1_v.at[0], acc1_sh.at[dst_v.at[0]],
                              ssem).wait()
        pltpu.make_async_copy(upd0_v.at[1], acc0_sh.at[dst_v.at[0]],
                              ssem).wait()
        pltpu.make_async_copy(upd1_v.at[1], acc1_sh.at[dst_v.at[0]],
                              ssem).wait()
        return carry
    lax.fori_loop(0, NSUP, outer, 0)
    plsc.subcore_barrier()
    pltpu.sync_copy(acc0_sh.at[pl.ds(s * RPS, RPS)],
                    out_hbm.at[pl.ds(c * 2 * NPAD + s * RPS, RPS)])
    pltpu.sync_copy(acc1_sh.at[pl.ds(s * RPS, RPS)],
                    out_hbm.at[pl.ds(c * 2 * NPAD + NPAD + s * RPS, RPS)])


_BLK = 1024
_GRID = NPAD // _BLK


def _tc1_body(x_ref, w1_ref, degp_ref, y1_ref, dinv_ref):
    deg = degp_ref[0, :] + degp_ref[1, :] + 1.0
    dinv = 1.0 / jnp.sqrt(deg)
    h = jnp.dot(x_ref[...], w1_ref[...], preferred_element_type=jnp.float32)
    y1_ref[...] = h * dinv[:, None]
    dinv_ref[...] = dinv


def _tc1(x, w1, degp):
    return pl.pallas_call(
        _tc1_body,
        grid=(_GRID,),
        in_specs=[
            pl.BlockSpec((_BLK, C), lambda i: (i, 0)),
            pl.BlockSpec((C, C), lambda i: (0, 0)),
            pl.BlockSpec((NC, _BLK), lambda i: (0, i)),
        ],
        out_specs=[
            pl.BlockSpec((_BLK, C), lambda i: (i, 0)),
            pl.BlockSpec((_BLK,), lambda i: (i,)),
        ],
        out_shape=[
            jax.ShapeDtypeStruct((NPAD, C), jnp.float32),
            jax.ShapeDtypeStruct((NPAD,), jnp.float32),
        ],
    )(x, w1, degp)


def _tc2_body(accp_ref, y1_ref, dinv_ref, b1_ref, w2_ref, y2t_ref):
    agg = accp_ref[0] + accp_ref[1] - y1_ref[...]
    dinv = dinv_ref[...]
    out1 = jax.nn.relu(agg * dinv[:, None] + b1_ref[...][None, :])
    y2t_ref[...] = lax.dot_general(
        w2_ref[...], out1, (((0,), (1,)), ((), ())),
        preferred_element_type=jnp.float32) * dinv[None, :]


def _tc2(accp, y1, dinv, b1, w2):
    return pl.pallas_call(
        _tc2_body,
        grid=(_GRID,),
        in_specs=[
            pl.BlockSpec((NC, _BLK, C), lambda i: (0, i, 0)),
            pl.BlockSpec((_BLK, C), lambda i: (i, 0)),
            pl.BlockSpec((_BLK,), lambda i: (i,)),
            pl.BlockSpec((C,), lambda i: (0,)),
            pl.BlockSpec((C, 2), lambda i: (0, 0)),
        ],
        out_specs=pl.BlockSpec((2, _BLK), lambda i: (0, i)),
        out_shape=jax.ShapeDtypeStruct((2, NPAD), jnp.float32),
    )(accp, y1, dinv, b1, w2)


def _tc3_body(acc2p_ref, y2_ref, dinv_ref, b2_ref, linw_ref, linb_ref,
              a1w_ref, a1b_ref, a2w_ref, a2b_ref, c1w_ref, c1b_ref,
              c2w_ref, c2b_ref, am_ref, sv_ref):
    agg = (acc2p_ref[0] + acc2p_ref[1] - y2_ref[...]).T
    dinv = dinv_ref[...]
    out2 = jax.nn.relu(agg * dinv[:, None] + b2_ref[...][None, :])
    feats = jnp.dot(out2, linw_ref[...],
                    preferred_element_type=jnp.float32) + linb_ref[...][None, :]
    a = jax.nn.relu(jnp.dot(feats, a1w_ref[...],
                            preferred_element_type=jnp.float32)
                    + a1b_ref[...][None, :])
    logits = jnp.dot(a, a2w_ref[...],
                     preferred_element_type=jnp.float32) + a2b_ref[...][None, :]
    m = jnp.max(logits, axis=-1, keepdims=True)
    ex = jnp.exp(logits - m)
    am_ref[...] = ex / jnp.sum(ex, axis=-1, keepdims=True)
    cv = jax.nn.relu(jnp.dot(feats, c1w_ref[...],
                             preferred_element_type=jnp.float32)
                     + c1b_ref[...][None, :])
    sv_ref[...] = jnp.dot(cv, c2w_ref[...],
                          preferred_element_type=jnp.float32) + c2b_ref[...][None, :]


def _tc3(acc2p, y2, dinv, b2, lin_w, lin_b, a1w, a1b, a2w, a2b, c1w, c1b,
         c2w, c2b):
    return pl.pallas_call(
        _tc3_body,
        grid=(_GRID,),
        in_specs=[
            pl.BlockSpec((NC, 2, _BLK), lambda i: (0, 0, i)),
            pl.BlockSpec((2, _BLK), lambda i: (0, i)),
            pl.BlockSpec((_BLK,), lambda i: (i,)),
            pl.BlockSpec((2,), lambda i: (0,)),
            pl.BlockSpec((2, 2), lambda i: (0, 0)),
            pl.BlockSpec((2,), lambda i: (0,)),
            pl.BlockSpec((2, C), lambda i: (0, 0)),
            pl.BlockSpec((C,), lambda i: (0,)),
            pl.BlockSpec((C, 10), lambda i: (0, 0)),
            pl.BlockSpec((10,), lambda i: (0,)),
            pl.BlockSpec((2, C), lambda i: (0, 0)),
            pl.BlockSpec((C,), lambda i: (0,)),
            pl.BlockSpec((C, 1), lambda i: (0, 0)),
            pl.BlockSpec((1,), lambda i: (0,)),
        ],
        out_specs=[
            pl.BlockSpec((_BLK, 10), lambda i: (i, 0)),
            pl.BlockSpec((_BLK, 1), lambda i: (i, 0)),
        ],
        out_shape=[
            jax.ShapeDtypeStruct((NPAD, 10), jnp.float32),
            jax.ShapeDtypeStruct((NPAD, 1), jnp.float32),
        ],
    )(acc2p, y2, dinv, b2, lin_w, lin_b, a1w, a1b, a2w, a2b, c1w, c1b,
      c2w, c2b)


def kernel(obs, edge_index, gcn_W1, gcn_b1, gcn_W2, gcn_b2, lin_W, lin_b,
           a1_W, a1_b, a2_W, a2_b, c1_W, c1_b, c2_W, c2_b):
    x = jnp.pad(obs[0], ((0, NPAD - N), (0, 0)))
    src = edge_index[0].astype(jnp.int32)
    dst = edge_index[1].astype(jnp.int32)
    npad_e = EP - E
    pad_i = jnp.arange(npad_e, dtype=jnp.int32)
    src_p = jnp.concatenate([src, (pad_i * 37) % N])
    dst_p = jnp.concatenate([dst, N + pad_i % TRASH])
    src_w = src_p.reshape(NW * NSUP, SUP, CHUNK)
    dst_w = dst_p.reshape(NW * NSUP, SUP, CHUNK)
    ones = jnp.ones((CHUNK,), jnp.float32)
    zeros = jnp.zeros((RPS,), jnp.float32)

    degp = _deg_kernel(dst_p.reshape(NW, NCHUNK, CHUNK), ones,
                       zeros).reshape(NC, NPAD)
    y1, dinv = _tc1(x, gcn_W1, degp)
    accp = _agg128(y1, src_w, dst_w).reshape(NC, NPAD, C)
    y2t = _tc2(accp, y1, dinv, gcn_b1, gcn_W2)
    src_w2 = src_p.reshape(NW * NSUP, SUP * CHUNK)
    acc2p = _agg2(y2t, src_w2, dst_w).reshape(NC, 2, NPAD)
    am, sv = _tc3(acc2p, y2t, dinv, gcn_b2, lin_W, lin_b, a1_W, a1_b,
                  a2_W, a2_b, c1_W, c1_b, c2_W, c2_b)
    return am[:N], sv[:N, 0]
